# baseline (speedup 1.0000x reference)
import functools
import jax
import jax.numpy as jnp
from jax import lax
from jax.experimental import pallas as pl
from jax.experimental.pallas import tpu as pltpu
from jax.experimental.pallas import tpu_sc as plsc

N_NODES = 10000
N_EDGES = 320000
DIM = 128
N_GRAPHS = 16

NC = 2
NS = 16
NP = 10240
CH = 128
BC = 8
NBLK = 10
EWP = NBLK * BC * CH
EPAD = NC * NS * EWP - N_EDGES
RPT = NP // NS

RB = 1024
GRID = NP // RB
SUB = RB // DIM

_mesh = plsc.VectorSubcoreMesh(
    core_axis_name="c", subcore_axis_name="s", num_cores=NC, num_subcores=NS)


def _zero16():
    return jnp.zeros((16,), jnp.float32)


_CH_OFFS = list(range(0, CH - 15, 16)) + ([CH - 16] if CH % 16 else [])


def _edge_pipeline(cid, sid, idx_streams, gstart, gwait, scatter):
    for hbm, vmem, _ in idx_streams:
        pltpu.sync_copy(hbm.at[cid, sid, 0], vmem.at[0])
    gstart(0, 0, 0)

    def _ifetch(g1, pn):
        for hbm, vmem, sem in idx_streams:
            pltpu.async_copy(hbm.at[cid, sid, g1], vmem.at[pn], sem)

    def _iwait(g1, pn):
        for hbm, vmem, sem in idx_streams:
            pltpu.make_async_copy(hbm.at[cid, sid, g1], vmem.at[pn],
                                  sem).wait()

    def _block(g, p, prefetch, last):
        pn = 1 - p
        if prefetch:
            _ifetch(g + 1, pn)
        for jb in range(BC):
            rb = jb % 2
            if jb == BC - 1:
                if not last:
                    _iwait(g + 1, pn)
                    gstart(pn, 0, 1 - rb)
            else:
                gstart(p, jb + 1, 1 - rb)
            gwait(p, jb, rb)
            scatter(p, jb, rb)

    def _two(t, c):
        _block(2 * t, 0, True, False)
        _block(2 * t + 1, 1, True, False)
        return c
    lax.fori_loop(0, (NBLK - 2) // 2, _two, 0)
    _block(NBLK - 2, 0, True, False)
    _block(NBLK - 1, 1, False, True)


def _sc_edge_body(with_deg, *refs):
    if with_deg:
        (p_hbm, src_hbm, dst_hbm, dstd_hbm, acc_out, deg_out,
         srcv, dstv, dstdv, rows, ones_v, acc_sh, deg_sh,
         gsem0, gsem1, ssem, isem_s, isem_d, isem_dd) = refs
    else:
        (p_hbm, src_hbm, dst_hbm, acc_out,
         srcv, dstv, rows, ones_v, acc_sh, deg_sh,
         gsem0, gsem1, ssem, isem_s, isem_d) = refs
        deg_out = None

    cid = lax.axis_index("c")
    sid = lax.axis_index("s")
    base = pl.multiple_of(sid * RPT, RPT)

    def _zrow(i, c):
        for k in range(DIM // 16):
            rows[0, i, pl.ds(k * 16, 16)] = _zero16()
        return c
    lax.fori_loop(0, CH, _zrow, 0)
    for o in _CH_OFFS:
        ones_v[pl.ds(o, 16)] = jnp.ones((16,), jnp.float32)
    for t in range(RPT // CH):
        pltpu.sync_copy(rows.at[0], acc_sh.at[pl.ds(base + t * CH, CH)])
    if with_deg:
        zvec = rows.at[0, 0]
        for t in range(RPT // DIM):
            pltpu.sync_copy(zvec, deg_sh.at[pl.ds(base + t * DIM, DIM)])
    plsc.subcore_barrier()

    def _gstart(pi, jb, b):
        pltpu.async_copy(p_hbm.at[srcv.at[pi, jb]], rows.at[b],
                         gsem0 if b == 0 else gsem1)

    def _gwait(pi, jb, b):
        pltpu.make_async_copy(p_hbm.at[srcv.at[pi, jb]], rows.at[b],
                              gsem0 if b == 0 else gsem1).wait()

    def _scatter(pi, jb, b):
        d = pltpu.async_copy(rows.at[b], acc_sh.at[dstv.at[pi, jb]], ssem,
                             add=True)
        if with_deg:
            pltpu.sync_copy(ones_v, deg_sh.at[dstdv.at[pi, jb]], add=True)
        d.wait()

    streams = [(src_hbm, srcv, isem_s), (dst_hbm, dstv, isem_d)]
    if with_deg:
        streams.append((dstd_hbm, dstdv, isem_dd))
    _edge_pipeline(cid, sid, streams, _gstart, _gwait, _scatter)
    plsc.subcore_barrier()

    pltpu.sync_copy(acc_sh.at[pl.ds(base, RPT)],
                    acc_out.at[cid, pl.ds(base, RPT)])
    if with_deg:
        pltpu.sync_copy(deg_sh.at[pl.ds(base, RPT)],
                        deg_out.at[cid, pl.ds(base, RPT)])


def _make_sc_edge(with_deg):
    out_type = [jax.ShapeDtypeStruct((NC, NP, DIM), jnp.float32)]
    if with_deg:
        out_type.append(jax.ShapeDtypeStruct((NC, NP), jnp.float32))
    scratch = [
        pltpu.VMEM((2, BC, CH), jnp.int32),
        pltpu.VMEM((2, BC, CH), jnp.int32),
    ]
    if with_deg:
        scratch.append(pltpu.VMEM((2, BC, CH), jnp.int32))
    scratch += [
        pltpu.VMEM((2, CH, DIM), jnp.float32),
        pltpu.VMEM((CH,), jnp.float32),
        pltpu.VMEM_SHARED((NP, DIM), jnp.float32),
        pltpu.VMEM_SHARED((NP,), jnp.float32),
        pltpu.SemaphoreType.DMA,
        pltpu.SemaphoreType.DMA,
        pltpu.SemaphoreType.DMA,
        pltpu.SemaphoreType.DMA,
        pltpu.SemaphoreType.DMA,
    ]
    if with_deg:
        scratch.append(pltpu.SemaphoreType.DMA)
    return pl.kernel(
        functools.partial(_sc_edge_body, with_deg),
        out_type=out_type,
        mesh=_mesh,
        scratch_types=scratch,
        name="sc_edge_segsum" + ("_deg" if with_deg else ""),
    )


_sc_edge_deg = _make_sc_edge(True)
_sc_edge = _make_sc_edge(False)


def _sc_scalar_body(q_hbm, src_hbm, dst_hbm, accq_out,
                    srcv, dstv, qrows, dacc, gsem0, gsem1, isem_s, isem_d):
    cid = lax.axis_index("c")
    sid = lax.axis_index("s")
    base = pl.multiple_of(sid * RPT, RPT)

    for o in _CH_OFFS:
        qrows[0, pl.ds(o, 16)] = _zero16()
    for t in range(RPT // CH):
        pltpu.sync_copy(qrows.at[0], dacc.at[pl.ds(base + t * CH, CH)])
    plsc.subcore_barrier()

    def _gstart(pi, jb, b):
        pltpu.async_copy(q_hbm.at[srcv.at[pi, jb]], qrows.at[b],
                         gsem0 if b == 0 else gsem1)

    def _gwait(pi, jb, b):
        pltpu.make_async_copy(q_hbm.at[srcv.at[pi, jb]], qrows.at[b],
                              gsem0 if b == 0 else gsem1).wait()

    def _scatter(pi, jb, b):
        pltpu.sync_copy(qrows.at[b], dacc.at[dstv.at[pi, jb]], add=True)

    _edge_pipeline(cid, sid,
                   [(src_hbm, srcv, isem_s), (dst_hbm, dstv, isem_d)],
                   _gstart, _gwait, _scatter)
    plsc.subcore_barrier()

    pltpu.sync_copy(dacc.at[pl.ds(base, RPT)],
                    accq_out.at[cid, pl.ds(base, RPT)])


_sc_scalar = pl.kernel(
    _sc_scalar_body,
    out_type=jax.ShapeDtypeStruct((NC, NP), jnp.float32),
    mesh=_mesh,
    scratch_types=[
        pltpu.VMEM((2, BC, CH), jnp.int32),
        pltpu.VMEM((2, BC, CH), jnp.int32),
        pltpu.VMEM((2, CH), jnp.float32),
        pltpu.VMEM_SHARED((NP,), jnp.float32),
        pltpu.SemaphoreType.DMA,
        pltpu.SemaphoreType.DMA,
        pltpu.SemaphoreType.DMA,
        pltpu.SemaphoreType.DMA,
    ],
    name="sc_scalar_segsum",
)


def _mm_body(x_ref, w_ref, o_ref):
    o_ref[...] = jnp.dot(x_ref[...], w_ref[...],
                         preferred_element_type=jnp.float32)


_mm = pl.pallas_call(
    _mm_body,
    grid=(GRID,),
    in_specs=[
        pl.BlockSpec((RB, DIM), lambda i: (i, 0)),
        pl.BlockSpec((DIM, DIM), lambda i: (0, 0)),
    ],
    out_specs=pl.BlockSpec((RB, DIM), lambda i: (i, 0)),
    out_shape=jax.ShapeDtypeStruct((NP, DIM), jnp.float32),
)


def _eye():
    return (lax.broadcasted_iota(jnp.int32, (DIM, DIM), 0)
            == lax.broadcasted_iota(jnp.int32, (DIM, DIM), 1)
            ).astype(jnp.float32)


def _cols_of(rows):
    return lax.dot_general(_eye(), rows, (((1,), (1,)), ((), ())),
                           preferred_element_type=jnp.float32)


def _rows_of(cols):
    return lax.dot_general(cols, _eye(), (((0,), (0,)), ((), ())),
                           preferred_element_type=jnp.float32)


def _tcb_body(acc_ref, degp_ref, x_ref, w1r_ref, w2l_ref, b1_ref,
              h1_ref, p2_ref, invd_ref):
    i = pl.program_id(0)
    deg = jnp.maximum(degp_ref[0] + degp_ref[1], 1.0)
    inv = 1.0 / deg
    invd_ref[...] = inv
    invT = _cols_of(inv)
    accs = acc_ref[0] + acc_ref[1]
    xr = (jnp.dot(x_ref[...], w1r_ref[...],
                  preferred_element_type=jnp.float32) + b1_ref[...])
    io0 = lax.broadcasted_iota(jnp.int32, (DIM, DIM), 0)
    for s in range(SUB):
        valid = (io0 + (RB * i + DIM * s) < N_NODES).astype(jnp.float32)
        mean_s = accs[s * DIM:(s + 1) * DIM, :] * invT[:, s:s + 1]
        h1_ref[pl.ds(s * DIM, DIM), :] = valid * jnp.maximum(
            mean_s + xr[s * DIM:(s + 1) * DIM, :], 0.0)
    p2_ref[...] = jnp.dot(h1_ref[...], w2l_ref[...],
                          preferred_element_type=jnp.float32)


_tcb = pl.pallas_call(
    _tcb_body,
    grid=(GRID,),
    in_specs=[
        pl.BlockSpec((NC, RB, DIM), lambda i: (0, i, 0)),
        pl.BlockSpec((NC, SUB, DIM), lambda i: (0, i, 0)),
        pl.BlockSpec((RB, DIM), lambda i: (i, 0)),
        pl.BlockSpec((DIM, DIM), lambda i: (0, 0)),
        pl.BlockSpec((DIM, DIM), lambda i: (0, 0)),
        pl.BlockSpec((1, DIM), lambda i: (0, 0)),
    ],
    out_specs=[
        pl.BlockSpec((RB, DIM), lambda i: (i, 0)),
        pl.BlockSpec((RB, DIM), lambda i: (i, 0)),
        pl.BlockSpec((SUB, DIM), lambda i: (i, 0)),
    ],
    out_shape=[
        jax.ShapeDtypeStruct((NP, DIM), jnp.float32),
        jax.ShapeDtypeStruct((NP, DIM), jnp.float32),
        jax.ShapeDtypeStruct((NP // DIM, DIM), jnp.float32),
    ],
)


def _tcc_body(acc_ref, invd_ref, h1_ref, w2r_ref, b2_ref, wp_ref, gi_ref,
              q_ref, rp_ref, gp_ref):
    i = pl.program_id(0)
    invT = _cols_of(invd_ref[...])
    giT = _cols_of(gi_ref[...].astype(jnp.float32))
    accs = acc_ref[0] + acc_ref[1]
    hr = (jnp.dot(h1_ref[...], w2r_ref[...],
                  preferred_element_type=jnp.float32) + b2_ref[...])
    io = lax.broadcasted_iota(jnp.int32, (DIM, N_GRAPHS), 1).astype(jnp.float32)
    h2_parts = []
    oh_parts = []
    for s in range(SUB):
        h2_s = (accs[s * DIM:(s + 1) * DIM, :] * invT[:, s:s + 1]
                + hr[s * DIM:(s + 1) * DIM, :])
        h2_parts.append(h2_s)
        oh_parts.append((giT[:, s:s + 1] == io).astype(jnp.float32))
    h2 = jnp.concatenate(h2_parts, axis=0)
    onehot = jnp.concatenate(oh_parts, axis=0)
    qrp = jnp.dot(h2, wp_ref[...], preferred_element_type=jnp.float32)
    q_cols = jnp.concatenate(
        [qrp[s * DIM:(s + 1) * DIM, 0:1] for s in range(SUB)], axis=1)
    r_cols = jnp.concatenate(
        [qrp[s * DIM:(s + 1) * DIM, 1:2] for s in range(SUB)], axis=1)
    node8 = (RB * i
             + DIM * lax.broadcasted_iota(jnp.int32, (SUB, DIM), 0)
             + lax.broadcasted_iota(jnp.int32, (SUB, DIM), 1))
    vm8 = (node8 < N_NODES).astype(jnp.float32)
    q_ref[...] = _rows_of(q_cols) * vm8
    rp_ref[...] = _rows_of(r_cols) * vm8
    part = lax.dot_general(onehot, h2, (((0,), (0,)), ((), ())),
                           preferred_element_type=jnp.float32)

    @pl.when(i == 0)
    def _():
        gp_ref[...] = part

    @pl.when(i > 0)
    def _():
        gp_ref[...] += part


_tcc = pl.pallas_call(
    _tcc_body,
    grid=(GRID,),
    in_specs=[
        pl.BlockSpec((NC, RB, DIM), lambda i: (0, i, 0)),
        pl.BlockSpec((SUB, DIM), lambda i: (i, 0)),
        pl.BlockSpec((RB, DIM), lambda i: (i, 0)),
        pl.BlockSpec((DIM, DIM), lambda i: (0, 0)),
        pl.BlockSpec((1, DIM), lambda i: (0, 0)),
        pl.BlockSpec((DIM, 2), lambda i: (0, 0)),
        pl.BlockSpec((SUB, DIM), lambda i: (i, 0)),
    ],
    out_specs=[
        pl.BlockSpec((SUB, DIM), lambda i: (i, 0)),
        pl.BlockSpec((SUB, DIM), lambda i: (i, 0)),
        pl.BlockSpec((N_GRAPHS, DIM), lambda i: (0, 0)),
    ],
    out_shape=[
        jax.ShapeDtypeStruct((NP // DIM, DIM), jnp.float32),
        jax.ShapeDtypeStruct((NP // DIM, DIM), jnp.float32),
        jax.ShapeDtypeStruct((N_GRAPHS, DIM), jnp.float32),
    ],
)


def _tcd_body(accq_ref, invd_ref, rp_ref, bp_ref, gp_ref, wv_ref, bv_ref,
              pol_ref, val_ref):
    accq = accq_ref[0] + accq_ref[1]
    pol_ref[...] = accq * invd_ref[...] + rp_ref[...] + bp_ref[...]
    v = jnp.sum(gp_ref[...] * wv_ref[...], axis=1, keepdims=True) + bv_ref[...]
    val_ref[...] = jnp.broadcast_to(jax.nn.sigmoid(v), (N_GRAPHS, DIM))


_tcd = pl.pallas_call(
    _tcd_body,
    grid=(1,),
    in_specs=[
        pl.BlockSpec((NC, NP // DIM, DIM), lambda i: (0, 0, 0)),
        pl.BlockSpec((NP // DIM, DIM), lambda i: (0, 0)),
        pl.BlockSpec((NP // DIM, DIM), lambda i: (0, 0)),
        pl.BlockSpec((1, 1), lambda i: (0, 0)),
        pl.BlockSpec((N_GRAPHS, DIM), lambda i: (0, 0)),
        pl.BlockSpec((1, DIM), lambda i: (0, 0)),
        pl.BlockSpec((1, 1), lambda i: (0, 0)),
    ],
    out_specs=[
        pl.BlockSpec((NP // DIM, DIM), lambda i: (0, 0)),
        pl.BlockSpec((N_GRAPHS, DIM), lambda i: (0, 0)),
    ],
    out_shape=[
        jax.ShapeDtypeStruct((NP // DIM, DIM), jnp.float32),
        jax.ShapeDtypeStruct((N_GRAPHS, DIM), jnp.float32),
    ],
)


def kernel(x, edge_index, graph_indices,
           W1l, W1r, b1, W2l, W2r, b2, Wpl, Wpr, bp, Wv, bv):
    xp = jnp.pad(x, ((0, NP - N_NODES), (0, 0)))
    ar = jnp.arange(EPAD, dtype=jnp.int32)
    src_r = jnp.concatenate(
        [edge_index[0], jnp.full((EPAD,), N_NODES, jnp.int32)]
    ).reshape(NC, NS, NBLK, BC, CH)
    dst_r = jnp.concatenate(
        [edge_index[1], ar % NP]
    ).reshape(NC, NS, NBLK, BC, CH)
    dstd_r = jnp.concatenate(
        [edge_index[1], N_NODES + ar % (NP - N_NODES)]
    ).reshape(NC, NS, NBLK, BC, CH)
    gi_pad = jnp.pad(graph_indices, (0, NP - N_NODES),
                     constant_values=N_GRAPHS).reshape(NP // DIM, DIM)
    b1r = b1.reshape(1, DIM)
    b2r = b2.reshape(1, DIM)
    wp = jnp.concatenate([Wpl, Wpr], axis=1)
    wv_row = Wv.reshape(1, DIM)
    bp_r = bp.reshape(1, 1)
    bv_r = bv.reshape(1, 1)

    p1 = _mm(xp, W1l)
    acc1, degp = _sc_edge_deg(p1, src_r, dst_r, dstd_r)
    degp_r = degp.reshape(NC, NP // DIM, DIM)
    h1, p2, invd = _tcb(acc1, degp_r, xp, W1r, W2l, b1r)
    (acc2,) = _sc_edge(p2, src_r, dst_r)
    q, rp, gp = _tcc(acc2, invd, h1, W2r, b2r, wp, gi_pad)
    accq = _sc_scalar(q.reshape(NP), src_r, dst_r)
    accq_r = accq.reshape(NC, NP // DIM, DIM)
    pol_r, val_b = _tcd(accq_r, invd, rp, bp_r, gp, wv_row, bv_r)
    policy = pol_r.reshape(NP, 1)[:N_NODES]
    value = val_b[:, 0:1]
    return (policy, value)


# speedup vs baseline: 2.8723x; 2.8723x over previous
import functools
import jax
import jax.numpy as jnp
from jax import lax
from jax.experimental import pallas as pl
from jax.experimental.pallas import tpu as pltpu
from jax.experimental.pallas import tpu_sc as plsc

N_NODES = 10000
N_EDGES = 320000
DIM = 128
N_GRAPHS = 16

NC = 2
NS = 16
NP = 10240
CH = 128
BC = 8
NBLK = 10
EWP = NBLK * BC * CH
EPAD = NC * NS * EWP - N_EDGES
RPT = NP // NS

RB = 1024
GRID = NP // RB
SUB = RB // DIM

_mesh = plsc.VectorSubcoreMesh(
    core_axis_name="c", subcore_axis_name="s", num_cores=NC, num_subcores=NS)


def _zero16():
    return jnp.zeros((16,), jnp.float32)


_CH_OFFS = list(range(0, CH - 15, 16)) + ([CH - 16] if CH % 16 else [])


def _edge_pipeline(cid, sid, idx_streams, gstart, gwait, scatter):
    for hbm, vmem, _ in idx_streams:
        pltpu.sync_copy(hbm.at[cid, sid, 0], vmem.at[0])
    gstart(0, 0, 0)

    def _ifetch(g1, pn):
        for hbm, vmem, sem in idx_streams:
            pltpu.async_copy(hbm.at[cid, sid, g1], vmem.at[pn], sem)

    def _iwait(g1, pn):
        for hbm, vmem, sem in idx_streams:
            pltpu.make_async_copy(hbm.at[cid, sid, g1], vmem.at[pn],
                                  sem).wait()

    def _block(g, p, prefetch, last):
        pn = 1 - p
        if prefetch:
            _ifetch(g + 1, pn)
        for jb in range(BC):
            rb = jb % 2
            if jb == BC - 1:
                if not last:
                    _iwait(g + 1, pn)
                    gstart(pn, 0, 1 - rb)
            else:
                gstart(p, jb + 1, 1 - rb)
            gwait(p, jb, rb)
            scatter(p, jb, rb)

    def _two(t, c):
        _block(2 * t, 0, True, False)
        _block(2 * t + 1, 1, True, False)
        return c
    lax.fori_loop(0, (NBLK - 2) // 2, _two, 0)
    _block(NBLK - 2, 0, True, False)
    _block(NBLK - 1, 1, False, True)


def _sc_edge_body(with_deg, *refs):
    if with_deg:
        (p_hbm, src_hbm, dst_hbm, dstd_hbm, acc_out, deg_out,
         srcv, dstv, dstdv, rows, ones_v, acc_sh, deg_sh,
         gsem0, gsem1, ssem, isem_s, isem_d, isem_dd) = refs
    else:
        (p_hbm, src_hbm, dst_hbm, acc_out,
         srcv, dstv, rows, ones_v, acc_sh, deg_sh,
         gsem0, gsem1, ssem, isem_s, isem_d) = refs
        deg_out = None

    cid = lax.axis_index("c")
    sid = lax.axis_index("s")
    base = pl.multiple_of(sid * RPT, RPT)

    def _zrow(i, c):
        for k in range(DIM // 16):
            rows[0, i, pl.ds(k * 16, 16)] = _zero16()
        return c
    lax.fori_loop(0, CH, _zrow, 0)
    for o in _CH_OFFS:
        ones_v[pl.ds(o, 16)] = jnp.ones((16,), jnp.float32)
    for t in range(RPT // CH):
        pltpu.sync_copy(rows.at[0], acc_sh.at[pl.ds(base + t * CH, CH)])
    if with_deg:
        zvec = rows.at[0, 0]
        for t in range(RPT // DIM):
            pltpu.sync_copy(zvec, deg_sh.at[pl.ds(base + t * DIM, DIM)])
    plsc.subcore_barrier()

    def _gstart(pi, jb, b):
        pltpu.async_copy(p_hbm.at[srcv.at[pi, jb]], rows.at[b],
                         gsem0 if b == 0 else gsem1)

    def _gwait(pi, jb, b):
        pltpu.make_async_copy(p_hbm.at[srcv.at[pi, jb]], rows.at[b],
                              gsem0 if b == 0 else gsem1).wait()

    def _scatter(pi, jb, b):
        d = pltpu.async_copy(rows.at[b], acc_sh.at[dstv.at[pi, jb]], ssem,
                             add=True)
        if with_deg:
            pltpu.sync_copy(ones_v, deg_sh.at[dstdv.at[pi, jb]], add=True)
        d.wait()

    streams = [(src_hbm, srcv, isem_s), (dst_hbm, dstv, isem_d)]
    if with_deg:
        streams.append((dstd_hbm, dstdv, isem_dd))
    _edge_pipeline(cid, sid, streams, _gstart, _gwait, _scatter)
    plsc.subcore_barrier()

    pltpu.sync_copy(acc_sh.at[pl.ds(base, RPT)],
                    acc_out.at[cid, pl.ds(base, RPT)])
    if with_deg:
        pltpu.sync_copy(deg_sh.at[pl.ds(base, RPT)],
                        deg_out.at[cid, pl.ds(base, RPT)])


def _make_sc_edge(with_deg):
    out_type = [jax.ShapeDtypeStruct((NC, NP, DIM), jnp.float32)]
    if with_deg:
        out_type.append(jax.ShapeDtypeStruct((NC, NP), jnp.float32))
    scratch = [
        pltpu.VMEM((2, BC, CH), jnp.int32),
        pltpu.VMEM((2, BC, CH), jnp.int32),
    ]
    if with_deg:
        scratch.append(pltpu.VMEM((2, BC, CH), jnp.int32))
    scratch += [
        pltpu.VMEM((2, CH, DIM), jnp.float32),
        pltpu.VMEM((CH,), jnp.float32),
        pltpu.VMEM_SHARED((NP, DIM), jnp.float32),
        pltpu.VMEM_SHARED((NP,), jnp.float32),
        pltpu.SemaphoreType.DMA,
        pltpu.SemaphoreType.DMA,
        pltpu.SemaphoreType.DMA,
        pltpu.SemaphoreType.DMA,
        pltpu.SemaphoreType.DMA,
    ]
    if with_deg:
        scratch.append(pltpu.SemaphoreType.DMA)
    return pl.kernel(
        functools.partial(_sc_edge_body, with_deg),
        out_type=out_type,
        mesh=_mesh,
        scratch_types=scratch,
        name="sc_edge_segsum" + ("_deg" if with_deg else ""),
    )


_sc_edge_deg = _make_sc_edge(True)
_sc_edge = _make_sc_edge(False)


def _sc_scalar_body(q_hbm, src_hbm, dst_hbm, accq_out,
                    srcv, dstv, qrows, dacc, gsem0, gsem1, isem_s, isem_d):
    cid = lax.axis_index("c")
    sid = lax.axis_index("s")
    base = pl.multiple_of(sid * RPT, RPT)

    for o in _CH_OFFS:
        qrows[0, pl.ds(o, 16)] = _zero16()
    for t in range(RPT // CH):
        pltpu.sync_copy(qrows.at[0], dacc.at[pl.ds(base + t * CH, CH)])
    plsc.subcore_barrier()

    def _gstart(pi, jb, b):
        pltpu.async_copy(q_hbm.at[srcv.at[pi, jb]], qrows.at[b],
                         gsem0 if b == 0 else gsem1)

    def _gwait(pi, jb, b):
        pltpu.make_async_copy(q_hbm.at[srcv.at[pi, jb]], qrows.at[b],
                              gsem0 if b == 0 else gsem1).wait()

    def _scatter(pi, jb, b):
        pltpu.sync_copy(qrows.at[b], dacc.at[dstv.at[pi, jb]], add=True)

    _edge_pipeline(cid, sid,
                   [(src_hbm, srcv, isem_s), (dst_hbm, dstv, isem_d)],
                   _gstart, _gwait, _scatter)
    plsc.subcore_barrier()

    pltpu.sync_copy(dacc.at[pl.ds(base, RPT)],
                    accq_out.at[cid, pl.ds(base, RPT)])


_sc_scalar = pl.kernel(
    _sc_scalar_body,
    out_type=jax.ShapeDtypeStruct((NC, NP), jnp.float32),
    mesh=_mesh,
    scratch_types=[
        pltpu.VMEM((2, BC, CH), jnp.int32),
        pltpu.VMEM((2, BC, CH), jnp.int32),
        pltpu.VMEM((2, CH), jnp.float32),
        pltpu.VMEM_SHARED((NP,), jnp.float32),
        pltpu.SemaphoreType.DMA,
        pltpu.SemaphoreType.DMA,
        pltpu.SemaphoreType.DMA,
        pltpu.SemaphoreType.DMA,
    ],
    name="sc_scalar_segsum",
)


def _mm_body(x_ref, w_ref, o_ref):
    o_ref[...] = jnp.dot(x_ref[...], w_ref[...],
                         preferred_element_type=jnp.float32)


_mm = pl.pallas_call(
    _mm_body,
    grid=(GRID,),
    in_specs=[
        pl.BlockSpec((RB, DIM), lambda i: (i, 0)),
        pl.BlockSpec((DIM, DIM), lambda i: (0, 0)),
    ],
    out_specs=pl.BlockSpec((RB, DIM), lambda i: (i, 0)),
    out_shape=jax.ShapeDtypeStruct((NP, DIM), jnp.float32),
)


def _eye():
    return (lax.broadcasted_iota(jnp.int32, (DIM, DIM), 0)
            == lax.broadcasted_iota(jnp.int32, (DIM, DIM), 1)
            ).astype(jnp.float32)


def _cols_of(rows):
    return lax.dot_general(_eye(), rows, (((1,), (1,)), ((), ())),
                           preferred_element_type=jnp.float32)


def _rows_of(cols):
    return lax.dot_general(cols, _eye(), (((0,), (0,)), ((), ())),
                           preferred_element_type=jnp.float32)


def _tcb_body(acc_ref, degp_ref, x_ref, w1r_ref, w2l_ref, b1_ref,
              h1_ref, p2_ref, invd_ref):
    i = pl.program_id(0)
    deg = jnp.maximum(degp_ref[0] + degp_ref[1], 1.0)
    inv = 1.0 / deg
    invd_ref[...] = inv
    invT = _cols_of(inv)
    accs = acc_ref[0] + acc_ref[1]
    xr = (jnp.dot(x_ref[...], w1r_ref[...],
                  preferred_element_type=jnp.float32) + b1_ref[...])
    io0 = lax.broadcasted_iota(jnp.int32, (DIM, DIM), 0)
    for s in range(SUB):
        valid = (io0 + (RB * i + DIM * s) < N_NODES).astype(jnp.float32)
        mean_s = accs[s * DIM:(s + 1) * DIM, :] * invT[:, s:s + 1]
        h1_ref[pl.ds(s * DIM, DIM), :] = valid * jnp.maximum(
            mean_s + xr[s * DIM:(s + 1) * DIM, :], 0.0)
    p2_ref[...] = jnp.dot(h1_ref[...], w2l_ref[...],
                          preferred_element_type=jnp.float32)


_tcb = pl.pallas_call(
    _tcb_body,
    grid=(GRID,),
    in_specs=[
        pl.BlockSpec((NC, RB, DIM), lambda i: (0, i, 0)),
        pl.BlockSpec((NC, SUB, DIM), lambda i: (0, i, 0)),
        pl.BlockSpec((RB, DIM), lambda i: (i, 0)),
        pl.BlockSpec((DIM, DIM), lambda i: (0, 0)),
        pl.BlockSpec((DIM, DIM), lambda i: (0, 0)),
        pl.BlockSpec((1, DIM), lambda i: (0, 0)),
    ],
    out_specs=[
        pl.BlockSpec((RB, DIM), lambda i: (i, 0)),
        pl.BlockSpec((RB, DIM), lambda i: (i, 0)),
        pl.BlockSpec((SUB, DIM), lambda i: (i, 0)),
    ],
    out_shape=[
        jax.ShapeDtypeStruct((NP, DIM), jnp.float32),
        jax.ShapeDtypeStruct((NP, DIM), jnp.float32),
        jax.ShapeDtypeStruct((NP // DIM, DIM), jnp.float32),
    ],
)


def _tcc_body(acc_ref, invd_ref, h1_ref, w2r_ref, b2_ref, wp_ref, gi_ref,
              q_ref, rp_ref, gp_ref):
    i = pl.program_id(0)
    invT = _cols_of(invd_ref[...])
    giT = _cols_of(gi_ref[...].astype(jnp.float32))
    accs = acc_ref[0] + acc_ref[1]
    hr = (jnp.dot(h1_ref[...], w2r_ref[...],
                  preferred_element_type=jnp.float32) + b2_ref[...])
    io = lax.broadcasted_iota(jnp.int32, (DIM, N_GRAPHS), 1).astype(jnp.float32)
    h2_parts = []
    oh_parts = []
    for s in range(SUB):
        h2_s = (accs[s * DIM:(s + 1) * DIM, :] * invT[:, s:s + 1]
                + hr[s * DIM:(s + 1) * DIM, :])
        h2_parts.append(h2_s)
        oh_parts.append((giT[:, s:s + 1] == io).astype(jnp.float32))
    h2 = jnp.concatenate(h2_parts, axis=0)
    onehot = jnp.concatenate(oh_parts, axis=0)
    qrp = jnp.dot(h2, wp_ref[...], preferred_element_type=jnp.float32)
    q_cols = jnp.concatenate(
        [qrp[s * DIM:(s + 1) * DIM, 0:1] for s in range(SUB)], axis=1)
    r_cols = jnp.concatenate(
        [qrp[s * DIM:(s + 1) * DIM, 1:2] for s in range(SUB)], axis=1)
    node8 = (RB * i
             + DIM * lax.broadcasted_iota(jnp.int32, (SUB, DIM), 0)
             + lax.broadcasted_iota(jnp.int32, (SUB, DIM), 1))
    vm8 = (node8 < N_NODES).astype(jnp.float32)
    q_ref[...] = _rows_of(q_cols) * vm8
    rp_ref[...] = _rows_of(r_cols) * vm8
    part = lax.dot_general(onehot, h2, (((0,), (0,)), ((), ())),
                           preferred_element_type=jnp.float32)

    @pl.when(i == 0)
    def _():
        gp_ref[...] = part

    @pl.when(i > 0)
    def _():
        gp_ref[...] += part


_tcc = pl.pallas_call(
    _tcc_body,
    grid=(GRID,),
    in_specs=[
        pl.BlockSpec((NC, RB, DIM), lambda i: (0, i, 0)),
        pl.BlockSpec((SUB, DIM), lambda i: (i, 0)),
        pl.BlockSpec((RB, DIM), lambda i: (i, 0)),
        pl.BlockSpec((DIM, DIM), lambda i: (0, 0)),
        pl.BlockSpec((1, DIM), lambda i: (0, 0)),
        pl.BlockSpec((DIM, 2), lambda i: (0, 0)),
        pl.BlockSpec((SUB, DIM), lambda i: (i, 0)),
    ],
    out_specs=[
        pl.BlockSpec((SUB, DIM), lambda i: (i, 0)),
        pl.BlockSpec((SUB, DIM), lambda i: (i, 0)),
        pl.BlockSpec((N_GRAPHS, DIM), lambda i: (0, 0)),
    ],
    out_shape=[
        jax.ShapeDtypeStruct((NP // DIM, DIM), jnp.float32),
        jax.ShapeDtypeStruct((NP // DIM, DIM), jnp.float32),
        jax.ShapeDtypeStruct((N_GRAPHS, DIM), jnp.float32),
    ],
)


def _tcd_body(accq_ref, invd_ref, rp_ref, bp_ref, gp_ref, wv_ref, bv_ref,
              pol_ref, val_ref):
    accq = accq_ref[0] + accq_ref[1]
    pol_ref[...] = accq * invd_ref[...] + rp_ref[...] + bp_ref[...]
    v = jnp.sum(gp_ref[...] * wv_ref[...], axis=1, keepdims=True) + bv_ref[...]
    val_ref[...] = jnp.broadcast_to(jax.nn.sigmoid(v), (N_GRAPHS, DIM))


_tcd = pl.pallas_call(
    _tcd_body,
    grid=(1,),
    in_specs=[
        pl.BlockSpec((NC, NP // DIM, DIM), lambda i: (0, 0, 0)),
        pl.BlockSpec((NP // DIM, DIM), lambda i: (0, 0)),
        pl.BlockSpec((NP // DIM, DIM), lambda i: (0, 0)),
        pl.BlockSpec((1, 1), lambda i: (0, 0)),
        pl.BlockSpec((N_GRAPHS, DIM), lambda i: (0, 0)),
        pl.BlockSpec((1, DIM), lambda i: (0, 0)),
        pl.BlockSpec((1, 1), lambda i: (0, 0)),
    ],
    out_specs=[
        pl.BlockSpec((NP // DIM, DIM), lambda i: (0, 0)),
        pl.BlockSpec((N_GRAPHS, DIM), lambda i: (0, 0)),
    ],
    out_shape=[
        jax.ShapeDtypeStruct((NP // DIM, DIM), jnp.float32),
        jax.ShapeDtypeStruct((N_GRAPHS, DIM), jnp.float32),
    ],
)


def kernel(x, edge_index, graph_indices,
           W1l, W1r, b1, W2l, W2r, b2, Wpl, Wpr, bp, Wv, bv):
    xp = jnp.pad(x, ((0, NP - N_NODES), (0, 0)))
    nw = NC * NS
    ew = N_EDGES // nw
    padw = EWP - ew
    ar = jnp.arange(EPAD, dtype=jnp.int32)
    pad_src = (N_NODES + ar % (NP - N_NODES)).reshape(nw, padw)
    pad_dst = (ar * 41) % NP
    pad_dstd = N_NODES + ar % (NP - N_NODES)

    def _tile_pack(real, pad):
        return jnp.concatenate(
            [real.reshape(nw, ew), pad.reshape(nw, padw)], axis=1
        ).reshape(NC, NS, NBLK, BC, CH)

    src_r = _tile_pack(edge_index[0], pad_src)
    dst_r = _tile_pack(edge_index[1], pad_dst)
    dstd_r = _tile_pack(edge_index[1], pad_dstd)
    gi_pad = jnp.pad(graph_indices, (0, NP - N_NODES),
                     constant_values=N_GRAPHS).reshape(NP // DIM, DIM)
    b1r = b1.reshape(1, DIM)
    b2r = b2.reshape(1, DIM)
    wp = jnp.concatenate([Wpl, Wpr], axis=1)
    wv_row = Wv.reshape(1, DIM)
    bp_r = bp.reshape(1, 1)
    bv_r = bv.reshape(1, 1)

    p1 = _mm(xp, W1l)
    acc1, degp = _sc_edge_deg(p1, src_r, dst_r, dstd_r)
    degp_r = degp.reshape(NC, NP // DIM, DIM)
    h1, p2, invd = _tcb(acc1, degp_r, xp, W1r, W2l, b1r)
    (acc2,) = _sc_edge(p2, src_r, dst_r)
    q, rp, gp = _tcc(acc2, invd, h1, W2r, b2r, wp, gi_pad)
    accq = _sc_scalar(q.reshape(NP), src_r, dst_r)
    accq_r = accq.reshape(NC, NP // DIM, DIM)
    pol_r, val_b = _tcd(accq_r, invd, rp, bp_r, gp, wv_row, bv_r)
    policy = pol_r.reshape(NP, 1)[:N_NODES]
    value = val_b[:, 0:1]
    return (policy, value)


# speedup vs baseline: 3.2279x; 1.1238x over previous
import functools
import jax
import jax.numpy as jnp
from jax import lax
from jax.experimental import pallas as pl
from jax.experimental.pallas import tpu as pltpu
from jax.experimental.pallas import tpu_sc as plsc

N_NODES = 10000
N_EDGES = 320000
DIM = 128
N_GRAPHS = 16

NC = 2
NS = 16
NP = 10240
CH = 128
BC = 8
NBLK = 10
EWP = NBLK * BC * CH
EPAD = NC * NS * EWP - N_EDGES
RPT = NP // NS

RB = 1024
GRID = NP // RB
SUB = RB // DIM

_mesh = plsc.VectorSubcoreMesh(
    core_axis_name="c", subcore_axis_name="s", num_cores=NC, num_subcores=NS)


def _zero16():
    return jnp.zeros((16,), jnp.float32)


_CH_OFFS = list(range(0, CH - 15, 16)) + ([CH - 16] if CH % 16 else [])


def _edge_pipeline(cid, sid, idx_streams, gstart, gwait, scatter):
    for hbm, vmem, _ in idx_streams:
        pltpu.sync_copy(hbm.at[cid, sid, 0], vmem.at[0])
    gstart(0, 0, 0)

    def _ifetch(g1, pn):
        for hbm, vmem, sem in idx_streams:
            pltpu.async_copy(hbm.at[cid, sid, g1], vmem.at[pn], sem)

    def _iwait(g1, pn):
        for hbm, vmem, sem in idx_streams:
            pltpu.make_async_copy(hbm.at[cid, sid, g1], vmem.at[pn],
                                  sem).wait()

    def _block(g, p, prefetch, last):
        pn = 1 - p
        if prefetch:
            _ifetch(g + 1, pn)
        for jb in range(BC):
            rb = jb % 2
            if jb == BC - 1:
                if not last:
                    _iwait(g + 1, pn)
                    gstart(pn, 0, 1 - rb)
            else:
                gstart(p, jb + 1, 1 - rb)
            gwait(p, jb, rb)
            scatter(p, jb, rb)

    def _two(t, c):
        _block(2 * t, 0, True, False)
        _block(2 * t + 1, 1, True, False)
        return c
    lax.fori_loop(0, (NBLK - 2) // 2, _two, 0)
    _block(NBLK - 2, 0, True, False)
    _block(NBLK - 1, 1, False, True)


def _sc_edge_body(with_deg, *refs):
    if with_deg:
        (p_hbm, src_hbm, dst_hbm, dstd_hbm, acc_out, deg_out,
         srcv, dstv, dstdv, rows, ones_v, acc_sh, deg_sh,
         gsem0, gsem1, ssem, isem_s, isem_d, isem_dd) = refs
    else:
        (p_hbm, src_hbm, dst_hbm, acc_out,
         srcv, dstv, rows, ones_v, acc_sh, deg_sh,
         gsem0, gsem1, ssem, isem_s, isem_d) = refs
        deg_out = None

    cid = lax.axis_index("c")
    sid = lax.axis_index("s")
    base = pl.multiple_of(sid * RPT, RPT)

    def _zrow(i, c):
        for k in range(DIM // 16):
            rows[0, i, pl.ds(k * 16, 16)] = _zero16()
        return c
    lax.fori_loop(0, CH, _zrow, 0)
    for o in _CH_OFFS:
        ones_v[pl.ds(o, 16)] = jnp.ones((16,), jnp.float32)
    for t in range(RPT // CH):
        pltpu.sync_copy(rows.at[0], acc_sh.at[pl.ds(base + t * CH, CH)])
    if with_deg:
        zvec = rows.at[0, 0]
        for t in range(RPT // DIM):
            pltpu.sync_copy(zvec, deg_sh.at[pl.ds(base + t * DIM, DIM)])
    plsc.subcore_barrier()

    def _gstart(pi, jb, b):
        pltpu.async_copy(p_hbm.at[srcv.at[pi, jb]], rows.at[b],
                         gsem0 if b == 0 else gsem1)

    def _gwait(pi, jb, b):
        pltpu.make_async_copy(p_hbm.at[srcv.at[pi, jb]], rows.at[b],
                              gsem0 if b == 0 else gsem1).wait()

    def _scatter(pi, jb, b):
        d = pltpu.async_copy(rows.at[b], acc_sh.at[dstv.at[pi, jb]], ssem,
                             add=True)
        if with_deg:
            pltpu.sync_copy(ones_v, deg_sh.at[dstdv.at[pi, jb]], add=True)
        d.wait()

    streams = [(src_hbm, srcv, isem_s), (dst_hbm, dstv, isem_d)]
    if with_deg:
        streams.append((dstd_hbm, dstdv, isem_dd))
    _edge_pipeline(cid, sid, streams, _gstart, _gwait, _scatter)
    plsc.subcore_barrier()

    pltpu.sync_copy(acc_sh.at[pl.ds(base, RPT)],
                    acc_out.at[cid, pl.ds(base, RPT)])
    if with_deg:
        pltpu.sync_copy(deg_sh.at[pl.ds(base, RPT)],
                        deg_out.at[cid, pl.ds(base, RPT)])


def _make_sc_edge(with_deg):
    out_type = [jax.ShapeDtypeStruct((NC, NP, DIM), jnp.float32)]
    if with_deg:
        out_type.append(jax.ShapeDtypeStruct((NC, NP), jnp.float32))
    scratch = [
        pltpu.VMEM((2, BC, CH), jnp.int32),
        pltpu.VMEM((2, BC, CH), jnp.int32),
    ]
    if with_deg:
        scratch.append(pltpu.VMEM((2, BC, CH), jnp.int32))
    scratch += [
        pltpu.VMEM((2, CH, DIM), jnp.float32),
        pltpu.VMEM((CH,), jnp.float32),
        pltpu.VMEM_SHARED((NP, DIM), jnp.float32),
        pltpu.VMEM_SHARED((NP,), jnp.float32),
        pltpu.SemaphoreType.DMA,
        pltpu.SemaphoreType.DMA,
        pltpu.SemaphoreType.DMA,
        pltpu.SemaphoreType.DMA,
        pltpu.SemaphoreType.DMA,
    ]
    if with_deg:
        scratch.append(pltpu.SemaphoreType.DMA)
    return pl.kernel(
        functools.partial(_sc_edge_body, with_deg),
        out_type=out_type,
        mesh=_mesh,
        scratch_types=scratch,
        name="sc_edge_segsum" + ("_deg" if with_deg else ""),
    )


_sc_edge_deg = _make_sc_edge(True)
_sc_edge = _make_sc_edge(False)


def _sc_scalar_body(q_hbm, src_hbm, dst_hbm, accq_out,
                    srcv, dstv, qv, qrows, dacc, isem_s, isem_d):
    cid = lax.axis_index("c")
    sid = lax.axis_index("s")
    base = pl.multiple_of(sid * RPT, RPT)

    pltpu.sync_copy(q_hbm, qv)
    for o in _CH_OFFS:
        qrows[pl.ds(o, 16)] = _zero16()
    for t in range(RPT // CH):
        pltpu.sync_copy(qrows, dacc.at[pl.ds(base + t * CH, CH)])
    plsc.subcore_barrier()

    def _gstart(pi, jb, b):
        del pi, jb, b

    def _gwait(pi, jb, b):
        del b
        for k in range(CH // 16):
            idx = srcv[pi, jb, pl.ds(k * 16, 16)]
            qrows[pl.ds(k * 16, 16)] = plsc.load_gather(qv, [idx])

    def _scatter(pi, jb, b):
        del b
        pltpu.sync_copy(qrows, dacc.at[dstv.at[pi, jb]], add=True)

    _edge_pipeline(cid, sid,
                   [(src_hbm, srcv, isem_s), (dst_hbm, dstv, isem_d)],
                   _gstart, _gwait, _scatter)
    plsc.subcore_barrier()

    pltpu.sync_copy(dacc.at[pl.ds(base, RPT)],
                    accq_out.at[cid, pl.ds(base, RPT)])


_sc_scalar = pl.kernel(
    _sc_scalar_body,
    out_type=jax.ShapeDtypeStruct((NC, NP), jnp.float32),
    mesh=_mesh,
    scratch_types=[
        pltpu.VMEM((2, BC, CH), jnp.int32),
        pltpu.VMEM((2, BC, CH), jnp.int32),
        pltpu.VMEM((NP,), jnp.float32),
        pltpu.VMEM((CH,), jnp.float32),
        pltpu.VMEM_SHARED((NP,), jnp.float32),
        pltpu.SemaphoreType.DMA,
        pltpu.SemaphoreType.DMA,
    ],
    compiler_params=pltpu.CompilerParams(needs_layout_passes=False),
    name="sc_scalar_segsum",
)


def _mm_body(x_ref, w_ref, o_ref):
    o_ref[...] = jnp.dot(x_ref[...], w_ref[...],
                         preferred_element_type=jnp.float32)


_mm = pl.pallas_call(
    _mm_body,
    grid=(GRID,),
    in_specs=[
        pl.BlockSpec((RB, DIM), lambda i: (i, 0)),
        pl.BlockSpec((DIM, DIM), lambda i: (0, 0)),
    ],
    out_specs=pl.BlockSpec((RB, DIM), lambda i: (i, 0)),
    out_shape=jax.ShapeDtypeStruct((NP, DIM), jnp.float32),
)


def _eye():
    return (lax.broadcasted_iota(jnp.int32, (DIM, DIM), 0)
            == lax.broadcasted_iota(jnp.int32, (DIM, DIM), 1)
            ).astype(jnp.float32)


def _cols_of(rows):
    return lax.dot_general(_eye(), rows, (((1,), (1,)), ((), ())),
                           preferred_element_type=jnp.float32)


def _rows_of(cols):
    return lax.dot_general(cols, _eye(), (((0,), (0,)), ((), ())),
                           preferred_element_type=jnp.float32)


def _tcb_body(acc_ref, degp_ref, x_ref, w1r_ref, w2l_ref, b1_ref,
              h1_ref, p2_ref, invd_ref):
    i = pl.program_id(0)
    deg = jnp.maximum(degp_ref[0] + degp_ref[1], 1.0)
    inv = 1.0 / deg
    invd_ref[...] = inv
    invT = _cols_of(inv)
    accs = acc_ref[0] + acc_ref[1]
    xr = (jnp.dot(x_ref[...], w1r_ref[...],
                  preferred_element_type=jnp.float32) + b1_ref[...])
    io0 = lax.broadcasted_iota(jnp.int32, (DIM, DIM), 0)
    for s in range(SUB):
        valid = (io0 + (RB * i + DIM * s) < N_NODES).astype(jnp.float32)
        mean_s = accs[s * DIM:(s + 1) * DIM, :] * invT[:, s:s + 1]
        h1_ref[pl.ds(s * DIM, DIM), :] = valid * jnp.maximum(
            mean_s + xr[s * DIM:(s + 1) * DIM, :], 0.0)
    p2_ref[...] = jnp.dot(h1_ref[...], w2l_ref[...],
                          preferred_element_type=jnp.float32)


_tcb = pl.pallas_call(
    _tcb_body,
    grid=(GRID,),
    in_specs=[
        pl.BlockSpec((NC, RB, DIM), lambda i: (0, i, 0)),
        pl.BlockSpec((NC, SUB, DIM), lambda i: (0, i, 0)),
        pl.BlockSpec((RB, DIM), lambda i: (i, 0)),
        pl.BlockSpec((DIM, DIM), lambda i: (0, 0)),
        pl.BlockSpec((DIM, DIM), lambda i: (0, 0)),
        pl.BlockSpec((1, DIM), lambda i: (0, 0)),
    ],
    out_specs=[
        pl.BlockSpec((RB, DIM), lambda i: (i, 0)),
        pl.BlockSpec((RB, DIM), lambda i: (i, 0)),
        pl.BlockSpec((SUB, DIM), lambda i: (i, 0)),
    ],
    out_shape=[
        jax.ShapeDtypeStruct((NP, DIM), jnp.float32),
        jax.ShapeDtypeStruct((NP, DIM), jnp.float32),
        jax.ShapeDtypeStruct((NP // DIM, DIM), jnp.float32),
    ],
)


def _tcc_body(acc_ref, invd_ref, h1_ref, w2r_ref, b2_ref, wp_ref, gi_ref,
              q_ref, rp_ref, gp_ref):
    i = pl.program_id(0)
    invT = _cols_of(invd_ref[...])
    giT = _cols_of(gi_ref[...].astype(jnp.float32))
    accs = acc_ref[0] + acc_ref[1]
    hr = (jnp.dot(h1_ref[...], w2r_ref[...],
                  preferred_element_type=jnp.float32) + b2_ref[...])
    io = lax.broadcasted_iota(jnp.int32, (DIM, N_GRAPHS), 1).astype(jnp.float32)
    h2_parts = []
    oh_parts = []
    for s in range(SUB):
        h2_s = (accs[s * DIM:(s + 1) * DIM, :] * invT[:, s:s + 1]
                + hr[s * DIM:(s + 1) * DIM, :])
        h2_parts.append(h2_s)
        oh_parts.append((giT[:, s:s + 1] == io).astype(jnp.float32))
    h2 = jnp.concatenate(h2_parts, axis=0)
    onehot = jnp.concatenate(oh_parts, axis=0)
    qrp = jnp.dot(h2, wp_ref[...], preferred_element_type=jnp.float32)
    q_cols = jnp.concatenate(
        [qrp[s * DIM:(s + 1) * DIM, 0:1] for s in range(SUB)], axis=1)
    r_cols = jnp.concatenate(
        [qrp[s * DIM:(s + 1) * DIM, 1:2] for s in range(SUB)], axis=1)
    node8 = (RB * i
             + DIM * lax.broadcasted_iota(jnp.int32, (SUB, DIM), 0)
             + lax.broadcasted_iota(jnp.int32, (SUB, DIM), 1))
    vm8 = (node8 < N_NODES).astype(jnp.float32)
    q_ref[...] = _rows_of(q_cols) * vm8
    rp_ref[...] = _rows_of(r_cols) * vm8
    part = lax.dot_general(onehot, h2, (((0,), (0,)), ((), ())),
                           preferred_element_type=jnp.float32)

    @pl.when(i == 0)
    def _():
        gp_ref[...] = part

    @pl.when(i > 0)
    def _():
        gp_ref[...] += part


_tcc = pl.pallas_call(
    _tcc_body,
    grid=(GRID,),
    in_specs=[
        pl.BlockSpec((NC, RB, DIM), lambda i: (0, i, 0)),
        pl.BlockSpec((SUB, DIM), lambda i: (i, 0)),
        pl.BlockSpec((RB, DIM), lambda i: (i, 0)),
        pl.BlockSpec((DIM, DIM), lambda i: (0, 0)),
        pl.BlockSpec((1, DIM), lambda i: (0, 0)),
        pl.BlockSpec((DIM, 2), lambda i: (0, 0)),
        pl.BlockSpec((SUB, DIM), lambda i: (i, 0)),
    ],
    out_specs=[
        pl.BlockSpec((SUB, DIM), lambda i: (i, 0)),
        pl.BlockSpec((SUB, DIM), lambda i: (i, 0)),
        pl.BlockSpec((N_GRAPHS, DIM), lambda i: (0, 0)),
    ],
    out_shape=[
        jax.ShapeDtypeStruct((NP // DIM, DIM), jnp.float32),
        jax.ShapeDtypeStruct((NP // DIM, DIM), jnp.float32),
        jax.ShapeDtypeStruct((N_GRAPHS, DIM), jnp.float32),
    ],
)


def _tcd_body(accq_ref, invd_ref, rp_ref, bp_ref, gp_ref, wv_ref, bv_ref,
              pol_ref, val_ref):
    accq = accq_ref[0] + accq_ref[1]
    pol_ref[...] = accq * invd_ref[...] + rp_ref[...] + bp_ref[...]
    v = jnp.sum(gp_ref[...] * wv_ref[...], axis=1, keepdims=True) + bv_ref[...]
    val_ref[...] = jnp.broadcast_to(jax.nn.sigmoid(v), (N_GRAPHS, DIM))


_tcd = pl.pallas_call(
    _tcd_body,
    grid=(1,),
    in_specs=[
        pl.BlockSpec((NC, NP // DIM, DIM), lambda i: (0, 0, 0)),
        pl.BlockSpec((NP // DIM, DIM), lambda i: (0, 0)),
        pl.BlockSpec((NP // DIM, DIM), lambda i: (0, 0)),
        pl.BlockSpec((1, 1), lambda i: (0, 0)),
        pl.BlockSpec((N_GRAPHS, DIM), lambda i: (0, 0)),
        pl.BlockSpec((1, DIM), lambda i: (0, 0)),
        pl.BlockSpec((1, 1), lambda i: (0, 0)),
    ],
    out_specs=[
        pl.BlockSpec((NP // DIM, DIM), lambda i: (0, 0)),
        pl.BlockSpec((N_GRAPHS, DIM), lambda i: (0, 0)),
    ],
    out_shape=[
        jax.ShapeDtypeStruct((NP // DIM, DIM), jnp.float32),
        jax.ShapeDtypeStruct((N_GRAPHS, DIM), jnp.float32),
    ],
)


def kernel(x, edge_index, graph_indices,
           W1l, W1r, b1, W2l, W2r, b2, Wpl, Wpr, bp, Wv, bv):
    xp = jnp.pad(x, ((0, NP - N_NODES), (0, 0)))
    nw = NC * NS
    ew = N_EDGES // nw
    padw = EWP - ew
    ar = jnp.arange(EPAD, dtype=jnp.int32)
    pad_src = (N_NODES + ar % (NP - N_NODES)).reshape(nw, padw)
    pad_dst = (ar * 41) % NP
    pad_dstd = N_NODES + ar % (NP - N_NODES)

    def _tile_pack(real, pad):
        return jnp.concatenate(
            [real.reshape(nw, ew), pad.reshape(nw, padw)], axis=1
        ).reshape(NC, NS, NBLK, BC, CH)

    src_r = _tile_pack(edge_index[0], pad_src)
    dst_r = _tile_pack(edge_index[1], pad_dst)
    dstd_r = _tile_pack(edge_index[1], pad_dstd)
    gi_pad = jnp.pad(graph_indices, (0, NP - N_NODES),
                     constant_values=N_GRAPHS).reshape(NP // DIM, DIM)
    b1r = b1.reshape(1, DIM)
    b2r = b2.reshape(1, DIM)
    wp = jnp.concatenate([Wpl, Wpr], axis=1)
    wv_row = Wv.reshape(1, DIM)
    bp_r = bp.reshape(1, 1)
    bv_r = bv.reshape(1, 1)

    p1 = _mm(xp, W1l)
    acc1, degp = _sc_edge_deg(p1, src_r, dst_r, dstd_r)
    degp_r = degp.reshape(NC, NP // DIM, DIM)
    h1, p2, invd = _tcb(acc1, degp_r, xp, W1r, W2l, b1r)
    (acc2,) = _sc_edge(p2, src_r, dst_r)
    q, rp, gp = _tcc(acc2, invd, h1, W2r, b2r, wp, gi_pad)
    accq = _sc_scalar(q.reshape(NP), src_r, dst_r)
    accq_r = accq.reshape(NC, NP // DIM, DIM)
    pol_r, val_b = _tcd(accq_r, invd, rp, bp_r, gp, wv_row, bv_r)
    policy = pol_r.reshape(NP, 1)[:N_NODES]
    value = val_b[:, 0:1]
    return (policy, value)


# speedup vs baseline: 3.3570x; 1.0400x over previous
import functools
import jax
import jax.numpy as jnp
import numpy as np
from jax import lax
from jax.experimental import pallas as pl
from jax.experimental.pallas import tpu as pltpu
from jax.experimental.pallas import tpu_sc as plsc

N_NODES = 10000
N_EDGES = 320000
DIM = 128
N_GRAPHS = 16

NC = 2
NS = 16
NW = NC * NS
NP = 10240
CH = 128
BC = 8
NBLK = 10
EWP = NBLK * BC * CH
EPAD = NW * EWP - N_EDGES
RPT = NP // NS
ECH = N_EDGES // CH
NGRP = ECH // BC
B9R0 = (NBLK - 1) * NW * BC
NREAL9 = NGRP - (NBLK - 1) * NW
NLEFT = ECH - NGRP * BC
NDUM = EPAD // CH

_AR = np.arange(EPAD, dtype=np.int32)
_DUM_SRC = (N_NODES + _AR % (NP - N_NODES)).reshape(NDUM, CH)
_DUM_DST = ((_AR * 41) % NP).astype(np.int32).reshape(NDUM, CH)
_DWGT = np.ones((NW, BC, CH), np.float32)
for _w in range(NREAL9, NW):
    for _jb in range(BC):
        if (_w - NREAL9) * BC + _jb >= NLEFT:
            _DWGT[_w, _jb, :] = 0.0

RB = 1024
GRID = NP // RB
SUB = RB // DIM

_mesh = plsc.VectorSubcoreMesh(
    core_axis_name="c", subcore_axis_name="s", num_cores=NC, num_subcores=NS)


def _zero16():
    return jnp.zeros((16,), jnp.float32)


_CH_OFFS = list(range(0, CH - 15, 16)) + ([CH - 16] if CH % 16 else [])


def _make_idx_stage(w, er, eb9_s, eb9_d, srcv, dstv, isem_s, isem_d,
                    deg_extra=None):
    def _pairs(g1, pn):
        if isinstance(g1, int) and g1 == NBLK - 1:
            out = [
                (eb9_s.at[w], srcv.at[pn], isem_s),
                (eb9_d.at[w], dstv.at[pn], isem_d),
            ]
            if deg_extra is not None:
                dwgt, dwv, isem_dd = deg_extra
                out.append((dwgt.at[w], dwv, isem_dd))
            return out
        base = pl.multiple_of((g1 * NW + w) * BC, BC)
        return [
            (er.at[0, pl.ds(base, BC)], srcv.at[pn], isem_s),
            (er.at[1, pl.ds(base, BC)], dstv.at[pn], isem_d),
        ]

    def _stage0():
        for s, d, _ in _pairs(0, 0):
            pltpu.sync_copy(s, d)

    def _ifetch(g1, pn):
        for s, d, sem in _pairs(g1, pn):
            pltpu.async_copy(s, d, sem)

    def _iwait(g1, pn):
        for s, d, sem in _pairs(g1, pn):
            pltpu.make_async_copy(s, d, sem).wait()

    return _stage0, _ifetch, _iwait


def _edge_pipeline(stage0, ifetch, iwait, gstart, gwait, scatter,
                   scatter_last=None):
    if scatter_last is None:
        scatter_last = scatter
    stage0()
    gstart(0, 0, 0)

    def _block(g, p, prefetch, last):
        pn = 1 - p
        if prefetch:
            ifetch(g + 1, pn)
        sc = scatter_last if last else scatter
        for jb in range(BC):
            rb = jb % 2
            if jb == BC - 1:
                if not last:
                    iwait(g + 1, pn)
                    gstart(pn, 0, 1 - rb)
            else:
                gstart(p, jb + 1, 1 - rb)
            gwait(p, jb, rb)
            sc(p, jb, rb)

    def _two(t, c):
        _block(2 * t, 0, True, False)
        _block(2 * t + 1, 1, True, False)
        return c
    lax.fori_loop(0, (NBLK - 2) // 2, _two, 0)
    _block(NBLK - 2, 0, True, False)
    _block(NBLK - 1, 1, False, True)


def _sc_edge_body(with_deg, *refs):
    if with_deg:
        (p_hbm, er, eb9_s, eb9_d, dwgt, acc_out, deg_out,
         srcv, dstv, dwv, rows, ones_v, acc_sh, deg_sh,
         gsem0, gsem1, ssem, isem_s, isem_d, isem_dd) = refs
    else:
        (p_hbm, er, eb9_s, eb9_d, acc_out,
         srcv, dstv, rows, ones_v, acc_sh, deg_sh,
         gsem0, gsem1, ssem, isem_s, isem_d) = refs
        deg_out = None

    cid = lax.axis_index("c")
    sid = lax.axis_index("s")
    w = cid * NS + sid
    base = pl.multiple_of(sid * RPT, RPT)

    def _zrow(i, c):
        for k in range(DIM // 16):
            rows[0, i, pl.ds(k * 16, 16)] = _zero16()
        return c
    lax.fori_loop(0, CH, _zrow, 0)
    for o in _CH_OFFS:
        ones_v[pl.ds(o, 16)] = jnp.ones((16,), jnp.float32)
    for t in range(RPT // CH):
        pltpu.sync_copy(rows.at[0], acc_sh.at[pl.ds(base + t * CH, CH)])
    if with_deg:
        zvec = rows.at[0, 0]
        for t in range(RPT // DIM):
            pltpu.sync_copy(zvec, deg_sh.at[pl.ds(base + t * DIM, DIM)])
    plsc.subcore_barrier()

    def _gstart(pi, jb, b):
        pltpu.async_copy(p_hbm.at[srcv.at[pi, jb]], rows.at[b],
                         gsem0 if b == 0 else gsem1)

    def _gwait(pi, jb, b):
        pltpu.make_async_copy(p_hbm.at[srcv.at[pi, jb]], rows.at[b],
                              gsem0 if b == 0 else gsem1).wait()

    def _scatter(pi, jb, b):
        d = pltpu.async_copy(rows.at[b], acc_sh.at[dstv.at[pi, jb]], ssem,
                             add=True)
        if with_deg:
            pltpu.sync_copy(ones_v, deg_sh.at[dstv.at[pi, jb]], add=True)
        d.wait()

    def _scatter_last(pi, jb, b):
        d = pltpu.async_copy(rows.at[b], acc_sh.at[dstv.at[pi, jb]], ssem,
                             add=True)
        if with_deg:
            pltpu.sync_copy(dwv.at[jb], deg_sh.at[dstv.at[pi, jb]], add=True)
        d.wait()

    deg_extra = (dwgt, dwv, isem_dd) if with_deg else None
    stage0, ifetch, iwait = _make_idx_stage(
        w, er, eb9_s, eb9_d, srcv, dstv, isem_s, isem_d, deg_extra)
    _edge_pipeline(stage0, ifetch, iwait, _gstart, _gwait, _scatter,
                   _scatter_last)
    plsc.subcore_barrier()

    pltpu.sync_copy(acc_sh.at[pl.ds(base, RPT)],
                    acc_out.at[cid, pl.ds(base, RPT)])
    if with_deg:
        pltpu.sync_copy(deg_sh.at[pl.ds(base, RPT)],
                        deg_out.at[cid, pl.ds(base, RPT)])


def _make_sc_edge(with_deg):
    out_type = [jax.ShapeDtypeStruct((NC, NP, DIM), jnp.float32)]
    if with_deg:
        out_type.append(jax.ShapeDtypeStruct((NC, NP), jnp.float32))
    scratch = [
        pltpu.VMEM((2, BC, CH), jnp.int32),
        pltpu.VMEM((2, BC, CH), jnp.int32),
    ]
    if with_deg:
        scratch.append(pltpu.VMEM((BC, CH), jnp.float32))
    scratch += [
        pltpu.VMEM((2, CH, DIM), jnp.float32),
        pltpu.VMEM((CH,), jnp.float32),
        pltpu.VMEM_SHARED((NP, DIM), jnp.float32),
        pltpu.VMEM_SHARED((NP,), jnp.float32),
        pltpu.SemaphoreType.DMA,
        pltpu.SemaphoreType.DMA,
        pltpu.SemaphoreType.DMA,
        pltpu.SemaphoreType.DMA,
        pltpu.SemaphoreType.DMA,
    ]
    if with_deg:
        scratch.append(pltpu.SemaphoreType.DMA)
    return pl.kernel(
        functools.partial(_sc_edge_body, with_deg),
        out_type=out_type,
        mesh=_mesh,
        scratch_types=scratch,
        name="sc_edge_segsum" + ("_deg" if with_deg else ""),
    )


_sc_edge_deg = _make_sc_edge(True)
_sc_edge = _make_sc_edge(False)


def _sc_scalar_body(q_hbm, er, eb9_s, eb9_d, accq_out,
                    srcv, dstv, qv, qrows, dacc, isem_s, isem_d):
    cid = lax.axis_index("c")
    sid = lax.axis_index("s")
    w = cid * NS + sid
    base = pl.multiple_of(sid * RPT, RPT)

    pltpu.sync_copy(q_hbm, qv)
    for o in _CH_OFFS:
        qrows[pl.ds(o, 16)] = _zero16()
    for t in range(RPT // CH):
        pltpu.sync_copy(qrows, dacc.at[pl.ds(base + t * CH, CH)])
    plsc.subcore_barrier()

    def _gstart(pi, jb, b):
        del pi, jb, b

    def _gwait(pi, jb, b):
        del b
        for k in range(CH // 16):
            idx = srcv[pi, jb, pl.ds(k * 16, 16)]
            qrows[pl.ds(k * 16, 16)] = plsc.load_gather(qv, [idx])

    def _scatter(pi, jb, b):
        del b
        pltpu.sync_copy(qrows, dacc.at[dstv.at[pi, jb]], add=True)

    stage0, ifetch, iwait = _make_idx_stage(
        w, er, eb9_s, eb9_d, srcv, dstv, isem_s, isem_d)
    _edge_pipeline(stage0, ifetch, iwait, _gstart, _gwait, _scatter)
    plsc.subcore_barrier()

    pltpu.sync_copy(dacc.at[pl.ds(base, RPT)],
                    accq_out.at[cid, pl.ds(base, RPT)])


_sc_scalar = pl.kernel(
    _sc_scalar_body,
    out_type=jax.ShapeDtypeStruct((NC, NP), jnp.float32),
    mesh=_mesh,
    scratch_types=[
        pltpu.VMEM((2, BC, CH), jnp.int32),
        pltpu.VMEM((2, BC, CH), jnp.int32),
        pltpu.VMEM((NP,), jnp.float32),
        pltpu.VMEM((CH,), jnp.float32),
        pltpu.VMEM_SHARED((NP,), jnp.float32),
        pltpu.SemaphoreType.DMA,
        pltpu.SemaphoreType.DMA,
    ],
    compiler_params=pltpu.CompilerParams(needs_layout_passes=False),
    name="sc_scalar_segsum",
)


def _mm_body(x_ref, w_ref, o_ref):
    o_ref[...] = jnp.dot(x_ref[...], w_ref[...],
                         preferred_element_type=jnp.float32)


_mm = pl.pallas_call(
    _mm_body,
    grid=(GRID,),
    in_specs=[
        pl.BlockSpec((RB, DIM), lambda i: (i, 0)),
        pl.BlockSpec((DIM, DIM), lambda i: (0, 0)),
    ],
    out_specs=pl.BlockSpec((RB, DIM), lambda i: (i, 0)),
    out_shape=jax.ShapeDtypeStruct((NP, DIM), jnp.float32),
)


def _eye():
    return (lax.broadcasted_iota(jnp.int32, (DIM, DIM), 0)
            == lax.broadcasted_iota(jnp.int32, (DIM, DIM), 1)
            ).astype(jnp.float32)


def _cols_of(rows):
    return lax.dot_general(_eye(), rows, (((1,), (1,)), ((), ())),
                           preferred_element_type=jnp.float32)


def _rows_of(cols):
    return lax.dot_general(cols, _eye(), (((0,), (0,)), ((), ())),
                           preferred_element_type=jnp.float32)


def _tcb_body(acc_ref, degp_ref, x_ref, w1r_ref, w2l_ref, b1_ref,
              h1_ref, p2_ref, invd_ref):
    i = pl.program_id(0)
    deg = jnp.maximum(degp_ref[0] + degp_ref[1], 1.0)
    inv = 1.0 / deg
    invd_ref[...] = inv
    invT = _cols_of(inv)
    accs = acc_ref[0] + acc_ref[1]
    xr = (jnp.dot(x_ref[...], w1r_ref[...],
                  preferred_element_type=jnp.float32) + b1_ref[...])
    io0 = lax.broadcasted_iota(jnp.int32, (DIM, DIM), 0)
    for s in range(SUB):
        valid = (io0 + (RB * i + DIM * s) < N_NODES).astype(jnp.float32)
        mean_s = accs[s * DIM:(s + 1) * DIM, :] * invT[:, s:s + 1]
        h1_ref[pl.ds(s * DIM, DIM), :] = valid * jnp.maximum(
            mean_s + xr[s * DIM:(s + 1) * DIM, :], 0.0)
    p2_ref[...] = jnp.dot(h1_ref[...], w2l_ref[...],
                          preferred_element_type=jnp.float32)


_tcb = pl.pallas_call(
    _tcb_body,
    grid=(GRID,),
    in_specs=[
        pl.BlockSpec((NC, RB, DIM), lambda i: (0, i, 0)),
        pl.BlockSpec((NC, SUB, DIM), lambda i: (0, i, 0)),
        pl.BlockSpec((RB, DIM), lambda i: (i, 0)),
        pl.BlockSpec((DIM, DIM), lambda i: (0, 0)),
        pl.BlockSpec((DIM, DIM), lambda i: (0, 0)),
        pl.BlockSpec((1, DIM), lambda i: (0, 0)),
    ],
    out_specs=[
        pl.BlockSpec((RB, DIM), lambda i: (i, 0)),
        pl.BlockSpec((RB, DIM), lambda i: (i, 0)),
        pl.BlockSpec((SUB, DIM), lambda i: (i, 0)),
    ],
    out_shape=[
        jax.ShapeDtypeStruct((NP, DIM), jnp.float32),
        jax.ShapeDtypeStruct((NP, DIM), jnp.float32),
        jax.ShapeDtypeStruct((NP // DIM, DIM), jnp.float32),
    ],
)


def _tcc_body(acc_ref, invd_ref, h1_ref, w2r_ref, b2_ref, wp_ref, gi_ref,
              q_ref, rp_ref, gp_ref):
    i = pl.program_id(0)
    invT = _cols_of(invd_ref[...])
    giT = _cols_of(gi_ref[...].astype(jnp.float32))
    accs = acc_ref[0] + acc_ref[1]
    hr = (jnp.dot(h1_ref[...], w2r_ref[...],
                  preferred_element_type=jnp.float32) + b2_ref[...])
    io = lax.broadcasted_iota(jnp.int32, (DIM, N_GRAPHS), 1).astype(jnp.float32)
    h2_parts = []
    oh_parts = []
    for s in range(SUB):
        h2_s = (accs[s * DIM:(s + 1) * DIM, :] * invT[:, s:s + 1]
                + hr[s * DIM:(s + 1) * DIM, :])
        h2_parts.append(h2_s)
        oh_parts.append((giT[:, s:s + 1] == io).astype(jnp.float32))
    h2 = jnp.concatenate(h2_parts, axis=0)
    onehot = jnp.concatenate(oh_parts, axis=0)
    qrp = jnp.dot(h2, wp_ref[...], preferred_element_type=jnp.float32)
    q_cols = jnp.concatenate(
        [qrp[s * DIM:(s + 1) * DIM, 0:1] for s in range(SUB)], axis=1)
    r_cols = jnp.concatenate(
        [qrp[s * DIM:(s + 1) * DIM, 1:2] for s in range(SUB)], axis=1)
    node8 = (RB * i
             + DIM * lax.broadcasted_iota(jnp.int32, (SUB, DIM), 0)
             + lax.broadcasted_iota(jnp.int32, (SUB, DIM), 1))
    vm8 = (node8 < N_NODES).astype(jnp.float32)
    q_ref[...] = _rows_of(q_cols) * vm8
    rp_ref[...] = _rows_of(r_cols) * vm8
    part = lax.dot_general(onehot, h2, (((0,), (0,)), ((), ())),
                           preferred_element_type=jnp.float32)

    @pl.when(i == 0)
    def _():
        gp_ref[...] = part

    @pl.when(i > 0)
    def _():
        gp_ref[...] += part


_tcc = pl.pallas_call(
    _tcc_body,
    grid=(GRID,),
    in_specs=[
        pl.BlockSpec((NC, RB, DIM), lambda i: (0, i, 0)),
        pl.BlockSpec((SUB, DIM), lambda i: (i, 0)),
        pl.BlockSpec((RB, DIM), lambda i: (i, 0)),
        pl.BlockSpec((DIM, DIM), lambda i: (0, 0)),
        pl.BlockSpec((1, DIM), lambda i: (0, 0)),
        pl.BlockSpec((DIM, 2), lambda i: (0, 0)),
        pl.BlockSpec((SUB, DIM), lambda i: (i, 0)),
    ],
    out_specs=[
        pl.BlockSpec((SUB, DIM), lambda i: (i, 0)),
        pl.BlockSpec((SUB, DIM), lambda i: (i, 0)),
        pl.BlockSpec((N_GRAPHS, DIM), lambda i: (0, 0)),
    ],
    out_shape=[
        jax.ShapeDtypeStruct((NP // DIM, DIM), jnp.float32),
        jax.ShapeDtypeStruct((NP // DIM, DIM), jnp.float32),
        jax.ShapeDtypeStruct((N_GRAPHS, DIM), jnp.float32),
    ],
)


def _tcd_body(accq_ref, invd_ref, rp_ref, bp_ref, gp_ref, wv_ref, bv_ref,
              pol_ref, val_ref):
    accq = accq_ref[0] + accq_ref[1]
    pol_ref[...] = accq * invd_ref[...] + rp_ref[...] + bp_ref[...]
    v = jnp.sum(gp_ref[...] * wv_ref[...], axis=1, keepdims=True) + bv_ref[...]
    val_ref[...] = jnp.broadcast_to(jax.nn.sigmoid(v), (N_GRAPHS, DIM))


_tcd = pl.pallas_call(
    _tcd_body,
    grid=(1,),
    in_specs=[
        pl.BlockSpec((NC, NP // DIM, DIM), lambda i: (0, 0, 0)),
        pl.BlockSpec((NP // DIM, DIM), lambda i: (0, 0)),
        pl.BlockSpec((NP // DIM, DIM), lambda i: (0, 0)),
        pl.BlockSpec((1, 1), lambda i: (0, 0)),
        pl.BlockSpec((N_GRAPHS, DIM), lambda i: (0, 0)),
        pl.BlockSpec((1, DIM), lambda i: (0, 0)),
        pl.BlockSpec((1, 1), lambda i: (0, 0)),
    ],
    out_specs=[
        pl.BlockSpec((NP // DIM, DIM), lambda i: (0, 0)),
        pl.BlockSpec((N_GRAPHS, DIM), lambda i: (0, 0)),
    ],
    out_shape=[
        jax.ShapeDtypeStruct((NP // DIM, DIM), jnp.float32),
        jax.ShapeDtypeStruct((N_GRAPHS, DIM), jnp.float32),
    ],
)


def kernel(x, edge_index, graph_indices,
           W1l, W1r, b1, W2l, W2r, b2, Wpl, Wpr, bp, Wv, bv):
    xp = jnp.pad(x, ((0, NP - N_NODES), (0, 0)))
    er = edge_index.reshape(2, ECH, CH)

    def _blk9(row, dum):
        synth = jnp.concatenate(
            [er[row, NGRP * BC:], jnp.asarray(dum)], axis=0
        ).reshape(NW - NREAL9, BC, CH)
        real9 = er[row, B9R0:NGRP * BC].reshape(NREAL9, BC, CH)
        return jnp.concatenate([real9, synth], axis=0)

    eb9_s = _blk9(0, _DUM_SRC)
    eb9_d = _blk9(1, _DUM_DST)
    gi_pad = jnp.pad(graph_indices, (0, NP - N_NODES),
                     constant_values=N_GRAPHS).reshape(NP // DIM, DIM)
    b1r = b1.reshape(1, DIM)
    b2r = b2.reshape(1, DIM)
    wp = jnp.concatenate([Wpl, Wpr], axis=1)
    wv_row = Wv.reshape(1, DIM)
    bp_r = bp.reshape(1, 1)
    bv_r = bv.reshape(1, 1)

    p1 = _mm(xp, W1l)
    acc1, degp = _sc_edge_deg(p1, er, eb9_s, eb9_d, jnp.asarray(_DWGT))
    degp_r = degp.reshape(NC, NP // DIM, DIM)
    h1, p2, invd = _tcb(acc1, degp_r, xp, W1r, W2l, b1r)
    (acc2,) = _sc_edge(p2, er, eb9_s, eb9_d)
    q, rp, gp = _tcc(acc2, invd, h1, W2r, b2r, wp, gi_pad)
    accq = _sc_scalar(q.reshape(NP), er, eb9_s, eb9_d)
    accq_r = accq.reshape(NC, NP // DIM, DIM)
    pol_r, val_b = _tcd(accq_r, invd, rp, bp_r, gp, wv_row, bv_r)
    policy = pol_r.reshape(NP, 1)[:N_NODES]
    value = val_b[:, 0:1]
    return (policy, value)
